# baseline (speedup 1.0000x reference)
import functools

import jax
import jax.numpy as jnp
from jax import lax
from jax.experimental import pallas as pl
from jax.experimental.pallas import tpu as pltpu
from jax.experimental.pallas import tpu_sc as plsc

N_ATOM = 10000
N_EDGE = 160000
N_SUB = 320000
DCAT = 256
DTE = 384
DELIN = 272
NC = 2
NS = 16
NW = NC * NS
SPW = N_SUB // NW
CB = 40
NCHUNK_B = SPW // CB
NPAIR_B = NCHUNK_B // 2
NSEG = 12416
PT = NSEG // NS
NRANGE = 26
RPC = NRANGE // NC
VFP = NRANGE * NSEG
SCH = 2000
TPS = N_SUB // NS
NSCH = TPS // SCH
KQ = 128
CAP = SCH + KQ + 32

_LOG1P = (
    9.099033060345e-08, 9.999914490033e-01, -4.998010985495e-01,
    3.313336586544e-01, -2.391897221371e-01, 1.647818875233e-01,
    -9.231230953049e-02, 3.441791151292e-02, -6.074752453026e-03,
)

_MESH = functools.partial(
    plsc.VectorSubcoreMesh,
    core_axis_name="c", subcore_axis_name="s", num_cores=NC, num_subcores=NS)


def _prep_atoms_body(af, w0, w1, t0, t1):
    a = af[...]
    t0[...] = jnp.dot(a, w0[...], preferred_element_type=jnp.float32)
    t1[...] = jnp.dot(a, w1[...], preferred_element_type=jnp.float32)


def _prep_edges_body(ef, dist, we, b, te):
    proj = jnp.dot(ef[...], we[...], preferred_element_type=jnp.float32) + b[...]
    dd = dist[...]
    dfac = jnp.exp(-(dd * dd) / 18.0)
    pad = jnp.zeros((proj.shape[0], DTE - DCAT - 1), jnp.float32)
    te[...] = jnp.concatenate([proj, dfac, pad], axis=1)


def _prep_ang_body(ang, wa, ap):
    ap[...] = jnp.dot(ang[...], wa[...], preferred_element_type=jnp.float32)


def _final_body(vf, ef, w1, b1, w2, b2, o):
    h = jnp.concatenate([vf[...], ef[...]], axis=1)
    h = h @ w1[...] + b1[...]
    h = h * jax.nn.sigmoid(h)
    o[...] = h @ w2[...] + b2[...]


def _sc_gather_act_body(t0_hbm, t1_hbm, te_hbm, ap_hbm, ipk_hbm, act_hbm,
                        idxv, t0v, t1v, tev, apv, actv,
                        sg0, sg1, ss0, ss1):
    wid = lax.axis_index("s") * NC + lax.axis_index("c")
    base_c = wid * NCHUNK_B

    def fire(k, p, sg):
        pltpu.sync_copy(ipk_hbm.at[base_c + k], idxv.at[p])
        off = (base_c + k) * CB
        pltpu.async_copy(t0_hbm.at[idxv.at[p, 0]], t0v.at[p], sg)
        pltpu.async_copy(t1_hbm.at[idxv.at[p, 1]], t1v.at[p], sg)
        pltpu.async_copy(te_hbm.at[idxv.at[p, 2]], tev.at[p], sg)
        pltpu.async_copy(ap_hbm.at[pl.ds(off, CB)], apv.at[p], sg)

    def drain_g(p, sg):
        pltpu.make_async_copy(t0_hbm.at[pl.ds(0, CB)], t0v.at[p], sg).wait()
        pltpu.make_async_copy(t1_hbm.at[pl.ds(0, CB)], t1v.at[p], sg).wait()
        pltpu.make_async_copy(te_hbm.at[pl.ds(0, CB)], tev.at[p], sg).wait()
        pltpu.make_async_copy(ap_hbm.at[pl.ds(0, CB)], apv.at[p], sg).wait()

    def drain_s(p, ss):
        pltpu.make_async_copy(actv.at[p], act_hbm.at[pl.ds(0, CB)], ss).wait()

    def compute(k, p):
        def row(j, cr):
            dfac = tev[p, j, pl.ds(DCAT, 16)][0]
            for v in range(8):
                lo = pl.ds(v * 16, 16)
                hi = pl.ds(128 + v * 16, 16)
                f = t0v[p, j, lo] + t1v[p, j, lo] + tev[p, j, lo] + apv[p, j, lo]
                s = t0v[p, j, hi] + t1v[p, j, hi] + tev[p, j, hi] + apv[p, j, hi]
                sig = dfac / (1.0 + jnp.exp(-f))
                t = jnp.exp(-jnp.abs(s))
                poly = jnp.full((16,), _LOG1P[8], jnp.float32)
                for c in _LOG1P[7::-1]:
                    poly = poly * t + c
                sp = jnp.maximum(s, 0.0) + poly
                actv[p, j, lo] = sig * sp
            return cr
        lax.fori_loop(0, CB, row, 0)
        pltpu.async_copy(actv.at[p], act_hbm.at[pl.ds((base_c + k) * CB, CB)],
                         ss0 if p == 0 else ss1)

    pltpu.async_copy(actv.at[0], act_hbm.at[pl.ds(N_SUB, CB)], ss0)
    pltpu.async_copy(actv.at[1], act_hbm.at[pl.ds(N_SUB, CB)], ss1)
    fire(0, 0, sg0)

    def pair(kk, carry):
        k0 = 2 * kk
        fire(k0 + 1, 1, sg1)
        drain_g(0, sg0)
        drain_s(0, ss0)
        compute(k0, 0)
        nxt = jnp.minimum(k0 + 2, NCHUNK_B - 1)
        fire(nxt, 0, sg0)
        drain_g(1, sg1)
        drain_s(1, ss1)
        compute(k0 + 1, 1)
        return carry

    lax.fori_loop(0, NPAIR_B, pair, 0)
    drain_g(0, sg0)
    drain_s(0, ss0)
    drain_s(1, ss1)


def _sc_scatter_body(act_hbm, sidx_hbm, vfp_hbm, accS, pos_list, loff_list,
                     sidxv, rowsv, loffsm, sem):
    cid = lax.axis_index("c")
    sid = lax.axis_index("s")
    tile_lo = sid * TPS
    row0 = sid * PT

    def range_body(r, carry):
        base = (cid * RPC + r) * NSEG

        def zrow(i, c):
            for v in range(8):
                rowsv[i, pl.ds(v * 16, 16)] = jnp.zeros((16,), jnp.float32)
            return c
        lax.fori_loop(0, KQ, zrow, 0)
        for i in range(6):
            pltpu.sync_copy(rowsv, accS.at[pl.ds(row0 + i * KQ, KQ)])
        pltpu.sync_copy(rowsv.at[pl.ds(0, 8)], accS.at[pl.ds(row0 + 6 * KQ, 8)])
        plsc.subcore_barrier()

        def scan_chunk(ch, _unused):
            pltpu.sync_copy(sidx_hbm.at[pl.ds(tile_lo + ch * SCH, SCH)], sidxv)

            def group(g, cnt):
                v16 = sidxv[pl.ds(g * 16, 16)]
                m = (v16 >= base) & (v16 < base + NSEG)
                iota16 = lax.iota(jnp.int32, 16)
                zero16 = jnp.zeros((16,), jnp.int32)
                one16 = jnp.full((16,), 1, jnp.int32)
                c15 = jnp.full((16,), 15, jnp.int32)
                x = jnp.where(m, one16, zero16)
                for k in (1, 2, 4, 8):
                    sh = x.at[jnp.maximum(iota16 - k, 0)].get(
                        mode='promise_in_bounds')
                    x = x + jnp.where(iota16 >= k, sh, zero16)
                lo = jnp.full((16,), -1, jnp.int32)
                tgt = iota16 + 1
                for step in (16, 8, 4, 2, 1):
                    cand = jnp.minimum(lo + step, c15)
                    pv = x.at[cand].get(mode='promise_in_bounds')
                    lo = jnp.where(pv < tgt, cand, lo)
                lane = jnp.minimum(lo + 1, c15)
                vl = v16.at[lane].get(mode='promise_in_bounds')
                pos_list[pl.ds(cnt, 16)] = (tile_lo + ch * SCH + g * 16) + lane
                loff_list[pl.ds(cnt, 16)] = vl - base
                return cnt + x[15]

            cnt = lax.fori_loop(0, SCH // 16, group, jnp.int32(0))

            zpos = jnp.zeros((16,), jnp.int32)
            tloff = jnp.full((16,), NSEG, jnp.int32)
            for i in range(KQ // 16):
                pos_list[pl.ds(cnt + i * 16, 16)] = zpos
                loff_list[pl.ds(cnt + i * 16, 16)] = tloff
            nq = cnt // KQ + 1

            def scat(q, c_):
                for i in range(KQ // 16):
                    loffsm[pl.ds(i * 16, 16)] = loff_list[pl.ds(q * KQ + i * 16, 16)]
                pltpu.async_copy(
                    act_hbm.at[pos_list.at[pl.ds(q * KQ, KQ)]], rowsv,
                    sem).wait()
                pltpu.sync_copy(rowsv, accS.at[loffsm], add=True)
                return c_

            lax.fori_loop(0, nq, scat, 0)
            return jnp.int32(0)

        lax.fori_loop(0, NSCH, scan_chunk, jnp.int32(0))
        plsc.subcore_barrier()
        pltpu.sync_copy(accS.at[pl.ds(row0, PT)],
                        vfp_hbm.at[pl.ds(base + row0, PT)])
        return carry

    lax.fori_loop(0, RPC, range_body, 0)


def kernel(atom_fea, edge_fea, sub_atom_idx, sub_edge_idx, sub_edge_ang,
           sub_index, distance, Wf, bf, Ws, bs, We1, be1, We2, be2):
    f32 = jnp.float32
    i32 = jnp.int32
    W0 = jnp.concatenate([Wf[0:128], Ws[0:128]], axis=1)
    W1 = jnp.concatenate([Wf[128:256], Ws[128:256]], axis=1)
    WE = jnp.concatenate([Wf[256:272], Ws[256:272]], axis=1)
    WA = jnp.concatenate([Wf[272:288], Ws[272:288]], axis=1)
    bcat = jnp.concatenate([bf, bs]).reshape(1, DCAT)
    i0 = sub_atom_idx[:, 0].astype(i32)
    i1 = sub_atom_idx[:, 1].astype(i32)
    eix = sub_edge_idx.astype(i32)
    sidx = sub_index.astype(i32)
    dist2 = distance.reshape(N_EDGE, 1)

    t0, t1 = pl.pallas_call(
        _prep_atoms_body,
        grid=(10,),
        in_specs=[pl.BlockSpec((1000, 128), lambda i: (i, 0)),
                  pl.BlockSpec((128, DCAT), lambda i: (0, 0)),
                  pl.BlockSpec((128, DCAT), lambda i: (0, 0))],
        out_specs=[pl.BlockSpec((1000, DCAT), lambda i: (i, 0)),
                   pl.BlockSpec((1000, DCAT), lambda i: (i, 0))],
        out_shape=[jax.ShapeDtypeStruct((N_ATOM, DCAT), f32)] * 2,
    )(atom_fea, W0, W1)

    te = pl.pallas_call(
        _prep_edges_body,
        grid=(80,),
        in_specs=[pl.BlockSpec((2000, 16), lambda i: (i, 0)),
                  pl.BlockSpec((2000, 1), lambda i: (i, 0)),
                  pl.BlockSpec((16, DCAT), lambda i: (0, 0)),
                  pl.BlockSpec((1, DCAT), lambda i: (0, 0))],
        out_specs=pl.BlockSpec((2000, DTE), lambda i: (i, 0)),
        out_shape=jax.ShapeDtypeStruct((N_EDGE, DTE), f32),
    )(edge_fea, dist2, WE, bcat)

    ap = pl.pallas_call(
        _prep_ang_body,
        grid=(160,),
        in_specs=[pl.BlockSpec((2000, 16), lambda i: (i, 0)),
                  pl.BlockSpec((16, DCAT), lambda i: (0, 0))],
        out_specs=pl.BlockSpec((2000, DCAT), lambda i: (i, 0)),
        out_shape=jax.ShapeDtypeStruct((N_SUB, DCAT), f32),
    )(sub_edge_ang, WA)

    ipk = jnp.stack([i0.reshape(-1, CB), i1.reshape(-1, CB),
                     eix.reshape(-1, CB)], axis=1)
    act = pl.kernel(
        _sc_gather_act_body,
        out_type=jax.ShapeDtypeStruct((N_SUB + CB, 128), f32),
        mesh=_MESH(),
        scratch_types=[
            pltpu.VMEM((2, 3, CB), i32),
            pltpu.VMEM((2, CB, DCAT), f32), pltpu.VMEM((2, CB, DCAT), f32),
            pltpu.VMEM((2, CB, DTE), f32), pltpu.VMEM((2, CB, DCAT), f32),
            pltpu.VMEM((2, CB, 128), f32),
            pltpu.SemaphoreType.DMA, pltpu.SemaphoreType.DMA,
            pltpu.SemaphoreType.DMA, pltpu.SemaphoreType.DMA,
        ],
    )(t0, t1, te, ap, ipk)

    vfp = pl.kernel(
        _sc_scatter_body,
        out_type=jax.ShapeDtypeStruct((VFP, 128), f32),
        mesh=_MESH(),
        scratch_types=[
            pltpu.VMEM_SHARED((NSEG + 16, 128), f32),
            pltpu.VMEM((CAP,), i32), pltpu.VMEM((CAP,), i32),
            pltpu.VMEM((SCH,), i32),
            pltpu.VMEM((KQ, 128), f32),
            pltpu.VMEM((KQ,), i32),
            pltpu.SemaphoreType.DMA,
        ],
    )(act, sidx)

    vf2 = vfp[:2 * N_EDGE].reshape(N_EDGE, DCAT)

    out = pl.pallas_call(
        _final_body,
        grid=(80,),
        in_specs=[pl.BlockSpec((2000, DCAT), lambda i: (i, 0)),
                  pl.BlockSpec((2000, 16), lambda i: (i, 0)),
                  pl.BlockSpec((DELIN, 128), lambda i: (0, 0)),
                  pl.BlockSpec((1, 128), lambda i: (0, 0)),
                  pl.BlockSpec((128, 32), lambda i: (0, 0)),
                  pl.BlockSpec((1, 32), lambda i: (0, 0))],
        out_specs=pl.BlockSpec((2000, 32), lambda i: (i, 0)),
        out_shape=jax.ShapeDtypeStruct((N_EDGE, 32), f32),
    )(vf2, edge_fea, We1, be1.reshape(1, 128), We2, be2.reshape(1, 32))
    return out


# speedup vs baseline: 1.2239x; 1.2239x over previous
import functools

import jax
import jax.numpy as jnp
from jax import lax
from jax.experimental import pallas as pl
from jax.experimental.pallas import tpu as pltpu
from jax.experimental.pallas import tpu_sc as plsc

N_ATOM = 10000
N_EDGE = 160000
N_SUB = 320000
DCAT = 256
DTE = 384
DELIN = 272
NC = 2
NS = 16
NW = NC * NS
SPW = N_SUB // NW
CB = 40
NCHUNK_B = SPW // CB
NPAIR_B = NCHUNK_B // 2
NSEG = 12416
PT = NSEG // NS
NRANGE = 26
RPC = NRANGE // NC
VFP = NRANGE * NSEG
SCH = 2000
TPS = N_SUB // NS
NSCH = TPS // SCH
KQ = 128
CAP = SCH + KQ + 32

_LOG1P = (
    9.099033060345e-08, 9.999914490033e-01, -4.998010985495e-01,
    3.313336586544e-01, -2.391897221371e-01, 1.647818875233e-01,
    -9.231230953049e-02, 3.441791151292e-02, -6.074752453026e-03,
)

_MESH = functools.partial(
    plsc.VectorSubcoreMesh,
    core_axis_name="c", subcore_axis_name="s", num_cores=NC, num_subcores=NS)


def _prep_atoms_body(af, w0, w1, t0, t1):
    a = af[...]
    t0[...] = jnp.dot(a, w0[...], preferred_element_type=jnp.float32)
    t1[...] = jnp.dot(a, w1[...], preferred_element_type=jnp.float32)


def _prep_edges_body(ef, dist, we, b, te):
    proj = jnp.dot(ef[...], we[...], preferred_element_type=jnp.float32) + b[...]
    dd = dist[...]
    dfac = jnp.exp(-(dd * dd) / 18.0)
    pad = jnp.zeros((proj.shape[0], DTE - DCAT - 1), jnp.float32)
    te[...] = jnp.concatenate([proj, dfac, pad], axis=1)


def _prep_ang_body(ang, wa, ap):
    ap[...] = jnp.dot(ang[...], wa[...], preferred_element_type=jnp.float32)


def _final_body(vf, ef, w1, b1, w2, b2, o):
    h = jnp.concatenate([vf[...], ef[...]], axis=1)
    h = h @ w1[...] + b1[...]
    h = h * jax.nn.sigmoid(h)
    o[...] = h @ w2[...] + b2[...]


def _sc_gather_act_body(t0_hbm, t1_hbm, te_hbm, ap_hbm, ipk_hbm, act_hbm,
                        idxv, t0v, t1v, tev, apv, actv,
                        sg0, sg1, ss0, ss1):
    wid = lax.axis_index("s") * NC + lax.axis_index("c")
    base_c = wid * NCHUNK_B

    def fire(k, p, sg):
        pltpu.sync_copy(ipk_hbm.at[base_c + k], idxv.at[p])
        off = (base_c + k) * CB
        pltpu.async_copy(t0_hbm.at[idxv.at[p, 0]], t0v.at[p], sg)
        pltpu.async_copy(t1_hbm.at[idxv.at[p, 1]], t1v.at[p], sg)
        pltpu.async_copy(te_hbm.at[idxv.at[p, 2]], tev.at[p], sg)
        pltpu.async_copy(ap_hbm.at[pl.ds(off, CB)], apv.at[p], sg)

    def drain_g(p, sg):
        pltpu.make_async_copy(t0_hbm.at[pl.ds(0, CB)], t0v.at[p], sg).wait()
        pltpu.make_async_copy(t1_hbm.at[pl.ds(0, CB)], t1v.at[p], sg).wait()
        pltpu.make_async_copy(te_hbm.at[pl.ds(0, CB)], tev.at[p], sg).wait()
        pltpu.make_async_copy(ap_hbm.at[pl.ds(0, CB)], apv.at[p], sg).wait()

    def drain_s(p, ss):
        pltpu.make_async_copy(actv.at[p], act_hbm.at[pl.ds(0, CB)], ss).wait()

    def compute(k, p):
        @plsc.parallel_loop(0, CB, unroll=2)
        def row(j):
            dfac = tev[p, j, pl.ds(DCAT, 16)][0]
            for v in range(8):
                lo = pl.ds(v * 16, 16)
                hi = pl.ds(128 + v * 16, 16)
                f = t0v[p, j, lo] + t1v[p, j, lo] + tev[p, j, lo] + apv[p, j, lo]
                s = t0v[p, j, hi] + t1v[p, j, hi] + tev[p, j, hi] + apv[p, j, hi]
                sig = dfac / (1.0 + jnp.exp(-f))
                t = jnp.exp(-jnp.abs(s))
                poly = jnp.full((16,), _LOG1P[8], jnp.float32)
                for c in _LOG1P[7::-1]:
                    poly = poly * t + c
                sp = jnp.maximum(s, 0.0) + poly
                actv[p, j, lo] = sig * sp
        pltpu.async_copy(actv.at[p], act_hbm.at[pl.ds((base_c + k) * CB, CB)],
                         ss0 if p == 0 else ss1)

    pltpu.async_copy(actv.at[0], act_hbm.at[pl.ds(N_SUB, CB)], ss0)
    pltpu.async_copy(actv.at[1], act_hbm.at[pl.ds(N_SUB, CB)], ss1)
    fire(0, 0, sg0)

    def pair(kk, carry):
        k0 = 2 * kk
        fire(k0 + 1, 1, sg1)
        drain_g(0, sg0)
        drain_s(0, ss0)
        compute(k0, 0)
        nxt = jnp.minimum(k0 + 2, NCHUNK_B - 1)
        fire(nxt, 0, sg0)
        drain_g(1, sg1)
        drain_s(1, ss1)
        compute(k0 + 1, 1)
        return carry

    lax.fori_loop(0, NPAIR_B, pair, 0)
    drain_g(0, sg0)
    drain_s(0, ss0)
    drain_s(1, ss1)


def _sc_scatter_body(act_hbm, sidx_hbm, vfp_hbm, accS, pos_list, loff_list,
                     sidxv, rowsv, loffsm, sem):
    cid = lax.axis_index("c")
    sid = lax.axis_index("s")
    tile_lo = sid * TPS
    row0 = sid * PT

    def range_body(r, carry):
        base = (cid * RPC + r) * NSEG

        def zrow(i, c):
            for v in range(8):
                rowsv[i, pl.ds(v * 16, 16)] = jnp.zeros((16,), jnp.float32)
            return c
        lax.fori_loop(0, KQ, zrow, 0)
        for i in range(6):
            pltpu.sync_copy(rowsv, accS.at[pl.ds(row0 + i * KQ, KQ)])
        pltpu.sync_copy(rowsv.at[pl.ds(0, 8)], accS.at[pl.ds(row0 + 6 * KQ, 8)])
        plsc.subcore_barrier()

        def scan_chunk(ch, _unused):
            pltpu.sync_copy(sidx_hbm.at[pl.ds(tile_lo + ch * SCH, SCH)], sidxv)

            def group(g, cnt):
                v16 = sidxv[pl.ds(g * 16, 16)]
                m = (v16 >= base) & (v16 < base + NSEG)
                iota16 = lax.iota(jnp.int32, 16)
                zero16 = jnp.zeros((16,), jnp.int32)
                one16 = jnp.full((16,), 1, jnp.int32)
                c15 = jnp.full((16,), 15, jnp.int32)
                x = jnp.where(m, one16, zero16)
                for k in (1, 2, 4, 8):
                    sh = x.at[jnp.maximum(iota16 - k, 0)].get(
                        mode='promise_in_bounds')
                    x = x + jnp.where(iota16 >= k, sh, zero16)
                lo = jnp.full((16,), -1, jnp.int32)
                tgt = iota16 + 1
                for step in (16, 8, 4, 2, 1):
                    cand = jnp.minimum(lo + step, c15)
                    pv = x.at[cand].get(mode='promise_in_bounds')
                    lo = jnp.where(pv < tgt, cand, lo)
                lane = jnp.minimum(lo + 1, c15)
                vl = v16.at[lane].get(mode='promise_in_bounds')
                pos_list[pl.ds(cnt, 16)] = (tile_lo + ch * SCH + g * 16) + lane
                loff_list[pl.ds(cnt, 16)] = vl - base
                return cnt + x[15]

            cnt = plsc.parallel_loop(
                0, SCH // 16, carry=jnp.int32(0), unroll=2)(group)

            zpos = jnp.zeros((16,), jnp.int32)
            tloff = jnp.full((16,), NSEG, jnp.int32)
            for i in range(KQ // 16):
                pos_list[pl.ds(cnt + i * 16, 16)] = zpos
                loff_list[pl.ds(cnt + i * 16, 16)] = tloff
            nq = cnt // KQ + 1

            def scat(q, c_):
                for i in range(KQ // 16):
                    loffsm[pl.ds(i * 16, 16)] = loff_list[pl.ds(q * KQ + i * 16, 16)]
                pltpu.async_copy(
                    act_hbm.at[pos_list.at[pl.ds(q * KQ, KQ)]], rowsv,
                    sem).wait()
                pltpu.sync_copy(rowsv, accS.at[loffsm], add=True)
                return c_

            lax.fori_loop(0, nq, scat, 0)
            return jnp.int32(0)

        lax.fori_loop(0, NSCH, scan_chunk, jnp.int32(0))
        plsc.subcore_barrier()
        pltpu.sync_copy(accS.at[pl.ds(row0, PT)],
                        vfp_hbm.at[pl.ds(base + row0, PT)])
        return carry

    lax.fori_loop(0, RPC, range_body, 0)


def kernel(atom_fea, edge_fea, sub_atom_idx, sub_edge_idx, sub_edge_ang,
           sub_index, distance, Wf, bf, Ws, bs, We1, be1, We2, be2):
    f32 = jnp.float32
    i32 = jnp.int32
    W0 = jnp.concatenate([Wf[0:128], Ws[0:128]], axis=1)
    W1 = jnp.concatenate([Wf[128:256], Ws[128:256]], axis=1)
    WE = jnp.concatenate([Wf[256:272], Ws[256:272]], axis=1)
    WA = jnp.concatenate([Wf[272:288], Ws[272:288]], axis=1)
    bcat = jnp.concatenate([bf, bs]).reshape(1, DCAT)
    i0 = sub_atom_idx[:, 0].astype(i32)
    i1 = sub_atom_idx[:, 1].astype(i32)
    eix = sub_edge_idx.astype(i32)
    sidx = sub_index.astype(i32)
    dist2 = distance.reshape(N_EDGE, 1)

    t0, t1 = pl.pallas_call(
        _prep_atoms_body,
        grid=(10,),
        in_specs=[pl.BlockSpec((1000, 128), lambda i: (i, 0)),
                  pl.BlockSpec((128, DCAT), lambda i: (0, 0)),
                  pl.BlockSpec((128, DCAT), lambda i: (0, 0))],
        out_specs=[pl.BlockSpec((1000, DCAT), lambda i: (i, 0)),
                   pl.BlockSpec((1000, DCAT), lambda i: (i, 0))],
        out_shape=[jax.ShapeDtypeStruct((N_ATOM, DCAT), f32)] * 2,
    )(atom_fea, W0, W1)

    te = pl.pallas_call(
        _prep_edges_body,
        grid=(80,),
        in_specs=[pl.BlockSpec((2000, 16), lambda i: (i, 0)),
                  pl.BlockSpec((2000, 1), lambda i: (i, 0)),
                  pl.BlockSpec((16, DCAT), lambda i: (0, 0)),
                  pl.BlockSpec((1, DCAT), lambda i: (0, 0))],
        out_specs=pl.BlockSpec((2000, DTE), lambda i: (i, 0)),
        out_shape=jax.ShapeDtypeStruct((N_EDGE, DTE), f32),
    )(edge_fea, dist2, WE, bcat)

    ap = pl.pallas_call(
        _prep_ang_body,
        grid=(160,),
        in_specs=[pl.BlockSpec((2000, 16), lambda i: (i, 0)),
                  pl.BlockSpec((16, DCAT), lambda i: (0, 0))],
        out_specs=pl.BlockSpec((2000, DCAT), lambda i: (i, 0)),
        out_shape=jax.ShapeDtypeStruct((N_SUB, DCAT), f32),
    )(sub_edge_ang, WA)

    ipk = jnp.stack([i0.reshape(-1, CB), i1.reshape(-1, CB),
                     eix.reshape(-1, CB)], axis=1)
    act = pl.kernel(
        _sc_gather_act_body,
        out_type=jax.ShapeDtypeStruct((N_SUB + CB, 128), f32),
        mesh=_MESH(),
        scratch_types=[
            pltpu.VMEM((2, 3, CB), i32),
            pltpu.VMEM((2, CB, DCAT), f32), pltpu.VMEM((2, CB, DCAT), f32),
            pltpu.VMEM((2, CB, DTE), f32), pltpu.VMEM((2, CB, DCAT), f32),
            pltpu.VMEM((2, CB, 128), f32),
            pltpu.SemaphoreType.DMA, pltpu.SemaphoreType.DMA,
            pltpu.SemaphoreType.DMA, pltpu.SemaphoreType.DMA,
        ],
    )(t0, t1, te, ap, ipk)

    vfp = pl.kernel(
        _sc_scatter_body,
        out_type=jax.ShapeDtypeStruct((VFP, 128), f32),
        mesh=_MESH(),
        scratch_types=[
            pltpu.VMEM_SHARED((NSEG + 16, 128), f32),
            pltpu.VMEM((CAP,), i32), pltpu.VMEM((CAP,), i32),
            pltpu.VMEM((SCH,), i32),
            pltpu.VMEM((KQ, 128), f32),
            pltpu.VMEM((KQ,), i32),
            pltpu.SemaphoreType.DMA,
        ],
    )(act, sidx)

    vf2 = vfp[:2 * N_EDGE].reshape(N_EDGE, DCAT)

    out = pl.pallas_call(
        _final_body,
        grid=(80,),
        in_specs=[pl.BlockSpec((2000, DCAT), lambda i: (i, 0)),
                  pl.BlockSpec((2000, 16), lambda i: (i, 0)),
                  pl.BlockSpec((DELIN, 128), lambda i: (0, 0)),
                  pl.BlockSpec((1, 128), lambda i: (0, 0)),
                  pl.BlockSpec((128, 32), lambda i: (0, 0)),
                  pl.BlockSpec((1, 32), lambda i: (0, 0))],
        out_specs=pl.BlockSpec((2000, 32), lambda i: (i, 0)),
        out_shape=jax.ShapeDtypeStruct((N_EDGE, 32), f32),
    )(vf2, edge_fea, We1, be1.reshape(1, 128), We2, be2.reshape(1, 32))
    return out


# speedup vs baseline: 2.3144x; 1.8910x over previous
import functools

import jax
import jax.numpy as jnp
from jax import lax
from jax.experimental import pallas as pl
from jax.experimental.pallas import tpu as pltpu
from jax.experimental.pallas import tpu_sc as plsc

N_ATOM = 10000
N_EDGE = 160000
N_SUB = 320000
DCAT = 256
DTE = 384
DELIN = 272
NC = 2
NS = 16
NW = NC * NS
SPW = N_SUB // NW
CB = 40
NCHUNK_B = SPW // CB
NPAIR_B = NCHUNK_B // 2
NSEG = 12416
PT = NSEG // NS
NRANGE = 26
RPC = NRANGE // NC
VFP = NRANGE * NSEG
SCH = 4000
TPS = N_SUB // NS
NSCH = TPS // SCH
KQ = 96
CAP = SCH + KQ + 32

_LOG1P = (
    9.099033060345e-08, 9.999914490033e-01, -4.998010985495e-01,
    3.313336586544e-01, -2.391897221371e-01, 1.647818875233e-01,
    -9.231230953049e-02, 3.441791151292e-02, -6.074752453026e-03,
)

_MESH = functools.partial(
    plsc.VectorSubcoreMesh,
    core_axis_name="c", subcore_axis_name="s", num_cores=NC, num_subcores=NS)


def _prep_atoms_body(af, w0, w1, t0, t1):
    a = af[...]
    t0[...] = jnp.dot(a, w0[...], preferred_element_type=jnp.float32)
    t1[...] = jnp.dot(a, w1[...], preferred_element_type=jnp.float32)


def _prep_edges_body(ef, dist, we, b, te):
    proj = jnp.dot(ef[...], we[...], preferred_element_type=jnp.float32) + b[...]
    dd = dist[...]
    dfac = jnp.exp(-(dd * dd) / 18.0)
    pad = jnp.zeros((proj.shape[0], DTE - DCAT - 1), jnp.float32)
    te[...] = jnp.concatenate([proj, dfac, pad], axis=1)


def _prep_ang_body(ang, wa, ap):
    ap[...] = jnp.dot(ang[...], wa[...], preferred_element_type=jnp.float32)


def _final_body(vf, ef, w1, b1, w2, b2, o):
    h = jnp.concatenate([vf[...], ef[...]], axis=1)
    h = h @ w1[...] + b1[...]
    h = h * jax.nn.sigmoid(h)
    o[...] = h @ w2[...] + b2[...]


def _sc_gather_act_body(t0_hbm, t1_hbm, te_hbm, ap_hbm, ipk_hbm, act_hbm,
                        idxv, t0v, t1v, tev, apv, actv,
                        sg0, sg1, ss0, ss1):
    wid = lax.axis_index("s") * NC + lax.axis_index("c")
    base_c = wid * NCHUNK_B

    def fire(k, p, sg):
        pltpu.sync_copy(ipk_hbm.at[base_c + k], idxv.at[p])
        off = (base_c + k) * CB
        pltpu.async_copy(t0_hbm.at[idxv.at[p, 0]], t0v.at[p], sg)
        pltpu.async_copy(t1_hbm.at[idxv.at[p, 1]], t1v.at[p], sg)
        pltpu.async_copy(te_hbm.at[idxv.at[p, 2]], tev.at[p], sg)
        pltpu.async_copy(ap_hbm.at[pl.ds(off, CB)], apv.at[p], sg)

    def drain_g(p, sg):
        pltpu.make_async_copy(t0_hbm.at[pl.ds(0, CB)], t0v.at[p], sg).wait()
        pltpu.make_async_copy(t1_hbm.at[pl.ds(0, CB)], t1v.at[p], sg).wait()
        pltpu.make_async_copy(te_hbm.at[pl.ds(0, CB)], tev.at[p], sg).wait()
        pltpu.make_async_copy(ap_hbm.at[pl.ds(0, CB)], apv.at[p], sg).wait()

    def drain_s(p, ss):
        pltpu.make_async_copy(actv.at[p], act_hbm.at[pl.ds(0, CB)], ss).wait()

    def compute(k, p):
        @plsc.parallel_loop(0, CB, unroll=2)
        def row(j):
            dfac = tev[p, j, pl.ds(DCAT, 16)][0]
            for v in range(8):
                lo = pl.ds(v * 16, 16)
                hi = pl.ds(128 + v * 16, 16)
                f = t0v[p, j, lo] + t1v[p, j, lo] + tev[p, j, lo] + apv[p, j, lo]
                s = t0v[p, j, hi] + t1v[p, j, hi] + tev[p, j, hi] + apv[p, j, hi]
                sig = dfac / (1.0 + jnp.exp(-f))
                t = jnp.exp(-jnp.abs(s))
                poly = jnp.full((16,), _LOG1P[8], jnp.float32)
                for c in _LOG1P[7::-1]:
                    poly = poly * t + c
                sp = jnp.maximum(s, 0.0) + poly
                actv[p, j, lo] = sig * sp
        pltpu.async_copy(actv.at[p], act_hbm.at[pl.ds((base_c + k) * CB, CB)],
                         ss0 if p == 0 else ss1)

    pltpu.async_copy(actv.at[0], act_hbm.at[pl.ds(N_SUB, CB)], ss0)
    pltpu.async_copy(actv.at[1], act_hbm.at[pl.ds(N_SUB, CB)], ss1)
    fire(0, 0, sg0)

    def pair(kk, carry):
        k0 = 2 * kk
        fire(k0 + 1, 1, sg1)
        drain_g(0, sg0)
        drain_s(0, ss0)
        compute(k0, 0)
        nxt = jnp.minimum(k0 + 2, NCHUNK_B - 1)
        fire(nxt, 0, sg0)
        drain_g(1, sg1)
        drain_s(1, ss1)
        compute(k0 + 1, 1)
        return carry

    lax.fori_loop(0, NPAIR_B, pair, 0)
    drain_g(0, sg0)
    drain_s(0, ss0)
    drain_s(1, ss1)


def _sc_scatter_body(act_hbm, sidx_hbm, vfp_hbm, accS, pos_list, loff_list,
                     sidxv, rowsv, loffsm, sem):
    cid = lax.axis_index("c")
    sid = lax.axis_index("s")
    tile_lo = sid * TPS
    row0 = sid * PT

    def range_body(r, carry):
        base = (cid * RPC + r) * NSEG

        def zrow(i, c):
            for v in range(8):
                rowsv[i, pl.ds(v * 16, 16)] = jnp.zeros((16,), jnp.float32)
            return c
        lax.fori_loop(0, KQ, zrow, 0)
        for i in range(6):
            pltpu.sync_copy(rowsv, accS.at[pl.ds(row0 + i * KQ, KQ)])
        pltpu.sync_copy(rowsv.at[pl.ds(0, 8)], accS.at[pl.ds(row0 + 6 * KQ, 8)])
        plsc.subcore_barrier()

        def scan_chunk(ch, _unused):
            pltpu.sync_copy(sidx_hbm.at[pl.ds(tile_lo + ch * SCH, SCH)], sidxv)

            def compact16(off):
                v16 = sidxv[pl.ds(off, 16)]
                m = (v16 >= base) & (v16 < base + NSEG)
                iota16 = lax.iota(jnp.int32, 16)
                zero16 = jnp.zeros((16,), jnp.int32)
                one16 = jnp.full((16,), 1, jnp.int32)
                c15 = jnp.full((16,), 15, jnp.int32)
                x = jnp.where(m, one16, zero16)
                for k in (1, 2, 4, 8):
                    sh = x.at[jnp.maximum(iota16 - k, 0)].get(
                        mode='promise_in_bounds')
                    x = x + jnp.where(iota16 >= k, sh, zero16)
                lo = jnp.full((16,), -1, jnp.int32)
                tgt = iota16 + 1
                for step in (16, 8, 4, 2, 1):
                    cand = jnp.minimum(lo + step, c15)
                    pv = x.at[cand].get(mode='promise_in_bounds')
                    lo = jnp.where(pv < tgt, cand, lo)
                lane = jnp.minimum(lo + 1, c15)
                vl = v16.at[lane].get(mode='promise_in_bounds')
                return (tile_lo + ch * SCH + off) + lane, vl - base, x[15]

            def group(g, cnt):
                pa, la, ka = compact16(g * 32)
                pb, lb, kb = compact16(g * 32 + 16)
                pos_list[pl.ds(cnt, 16)] = pa
                loff_list[pl.ds(cnt, 16)] = la
                pos_list[pl.ds(cnt + ka, 16)] = pb
                loff_list[pl.ds(cnt + ka, 16)] = lb
                return cnt + ka + kb

            cnt = lax.fori_loop(0, SCH // 32, group, jnp.int32(0))

            zpos = jnp.zeros((16,), jnp.int32)
            tloff = jnp.full((16,), NSEG, jnp.int32)
            for i in range(KQ // 16):
                pos_list[pl.ds(cnt + i * 16, 16)] = zpos
                loff_list[pl.ds(cnt + i * 16, 16)] = tloff
            nq = cnt // KQ + 1

            def scat(q, c_):
                for i in range(KQ // 16):
                    loffsm[pl.ds(i * 16, 16)] = loff_list[pl.ds(q * KQ + i * 16, 16)]
                pltpu.async_copy(
                    act_hbm.at[pos_list.at[pl.ds(q * KQ, KQ)]], rowsv,
                    sem).wait()
                pltpu.sync_copy(rowsv, accS.at[loffsm], add=True)
                return c_

            lax.fori_loop(0, nq, scat, 0)
            return jnp.int32(0)

        lax.fori_loop(0, NSCH, scan_chunk, jnp.int32(0))
        plsc.subcore_barrier()
        pltpu.sync_copy(accS.at[pl.ds(row0, PT)],
                        vfp_hbm.at[pl.ds(base + row0, PT)])
        return carry

    lax.fori_loop(0, RPC, range_body, 0)


def kernel(atom_fea, edge_fea, sub_atom_idx, sub_edge_idx, sub_edge_ang,
           sub_index, distance, Wf, bf, Ws, bs, We1, be1, We2, be2):
    f32 = jnp.float32
    i32 = jnp.int32
    W0 = jnp.concatenate([Wf[0:128], Ws[0:128]], axis=1)
    W1 = jnp.concatenate([Wf[128:256], Ws[128:256]], axis=1)
    WE = jnp.concatenate([Wf[256:272], Ws[256:272]], axis=1)
    WA = jnp.concatenate([Wf[272:288], Ws[272:288]], axis=1)
    bcat = jnp.concatenate([bf, bs]).reshape(1, DCAT)
    i0 = sub_atom_idx[:, 0].astype(i32)
    i1 = sub_atom_idx[:, 1].astype(i32)
    eix = sub_edge_idx.astype(i32)
    sidx = sub_index.astype(i32)
    dist2 = distance.reshape(N_EDGE, 1)

    t0, t1 = pl.pallas_call(
        _prep_atoms_body,
        grid=(10,),
        in_specs=[pl.BlockSpec((1000, 128), lambda i: (i, 0)),
                  pl.BlockSpec((128, DCAT), lambda i: (0, 0)),
                  pl.BlockSpec((128, DCAT), lambda i: (0, 0))],
        out_specs=[pl.BlockSpec((1000, DCAT), lambda i: (i, 0)),
                   pl.BlockSpec((1000, DCAT), lambda i: (i, 0))],
        out_shape=[jax.ShapeDtypeStruct((N_ATOM, DCAT), f32)] * 2,
    )(atom_fea, W0, W1)

    te = pl.pallas_call(
        _prep_edges_body,
        grid=(80,),
        in_specs=[pl.BlockSpec((2000, 16), lambda i: (i, 0)),
                  pl.BlockSpec((2000, 1), lambda i: (i, 0)),
                  pl.BlockSpec((16, DCAT), lambda i: (0, 0)),
                  pl.BlockSpec((1, DCAT), lambda i: (0, 0))],
        out_specs=pl.BlockSpec((2000, DTE), lambda i: (i, 0)),
        out_shape=jax.ShapeDtypeStruct((N_EDGE, DTE), f32),
    )(edge_fea, dist2, WE, bcat)

    ap = pl.pallas_call(
        _prep_ang_body,
        grid=(160,),
        in_specs=[pl.BlockSpec((2000, 16), lambda i: (i, 0)),
                  pl.BlockSpec((16, DCAT), lambda i: (0, 0))],
        out_specs=pl.BlockSpec((2000, DCAT), lambda i: (i, 0)),
        out_shape=jax.ShapeDtypeStruct((N_SUB, DCAT), f32),
    )(sub_edge_ang, WA)

    ipk = jnp.stack([i0.reshape(-1, CB), i1.reshape(-1, CB),
                     eix.reshape(-1, CB)], axis=1)
    act = pl.kernel(
        _sc_gather_act_body,
        out_type=jax.ShapeDtypeStruct((N_SUB + CB, 128), f32),
        mesh=_MESH(),
        scratch_types=[
            pltpu.VMEM((2, 3, CB), i32),
            pltpu.VMEM((2, CB, DCAT), f32), pltpu.VMEM((2, CB, DCAT), f32),
            pltpu.VMEM((2, CB, DTE), f32), pltpu.VMEM((2, CB, DCAT), f32),
            pltpu.VMEM((2, CB, 128), f32),
            pltpu.SemaphoreType.DMA, pltpu.SemaphoreType.DMA,
            pltpu.SemaphoreType.DMA, pltpu.SemaphoreType.DMA,
        ],
    )(t0, t1, te, ap, ipk)

    vfp = pl.kernel(
        _sc_scatter_body,
        out_type=jax.ShapeDtypeStruct((VFP, 128), f32),
        mesh=_MESH(),
        scratch_types=[
            pltpu.VMEM_SHARED((NSEG + 16, 128), f32),
            pltpu.VMEM((CAP,), i32), pltpu.VMEM((CAP,), i32),
            pltpu.VMEM((SCH,), i32),
            pltpu.VMEM((KQ, 128), f32),
            pltpu.VMEM((KQ,), i32),
            pltpu.SemaphoreType.DMA,
        ],
    )(act, sidx)

    vf2 = vfp[:2 * N_EDGE].reshape(N_EDGE, DCAT)

    out = pl.pallas_call(
        _final_body,
        grid=(80,),
        in_specs=[pl.BlockSpec((2000, DCAT), lambda i: (i, 0)),
                  pl.BlockSpec((2000, 16), lambda i: (i, 0)),
                  pl.BlockSpec((DELIN, 128), lambda i: (0, 0)),
                  pl.BlockSpec((1, 128), lambda i: (0, 0)),
                  pl.BlockSpec((128, 32), lambda i: (0, 0)),
                  pl.BlockSpec((1, 32), lambda i: (0, 0))],
        out_specs=pl.BlockSpec((2000, 32), lambda i: (i, 0)),
        out_shape=jax.ShapeDtypeStruct((N_EDGE, 32), f32),
    )(vf2, edge_fea, We1, be1.reshape(1, 128), We2, be2.reshape(1, 32))
    return out
